# Initial kernel scaffold; baseline (speedup 1.0000x reference)
import jax
import jax.numpy as jnp
from jax.experimental import pallas as pl


def kernel(x, edge_attr, ne_W1, ne_b1, ne_a, ne_W2, ne_b2, ee_W1, ee_b1, ee_a, ee_W2, ee_b2, pr_W1, pr_b1, pr_a, pr_W2, pr_b2, eps, lp_W, lp_b, edge_index):
    raise NotImplementedError("write your pallas kernel here")


# speedup vs baseline: 2.0833x; 2.0833x over previous
import functools

import jax
import jax.numpy as jnp
from jax import lax
from jax.experimental import pallas as pl
from jax.experimental.pallas import tpu as pltpu
from jax.experimental.pallas import tpu_sc as plsc


def _prelu(h, a):
    return jnp.where(h >= 0, h, a * h)


def _mlp_body(x_ref, w1_ref, b1_ref, a_ref, w2_ref, b2_ref, o_ref):
    h = jnp.dot(x_ref[...], w1_ref[...], preferred_element_type=jnp.float32)
    h = _prelu(h + b1_ref[...], a_ref[0, 0])
    o_ref[...] = (
        jnp.dot(h, w2_ref[...], preferred_element_type=jnp.float32) + b2_ref[...]
    )


def _mlp_pallas(x, w1, b1, a, w2, b2, block_rows):
    n, _ = x.shape
    h_out = w2.shape[1]
    grid = n // block_rows
    full = lambda shape: pl.BlockSpec(shape, lambda i: (0, 0))
    return pl.pallas_call(
        _mlp_body,
        grid=(grid,),
        in_specs=[
            pl.BlockSpec((block_rows, x.shape[1]), lambda i: (i, 0)),
            full(w1.shape),
            full((1, w1.shape[1])),
            full((1, 1)),
            full(w2.shape),
            full((1, h_out)),
        ],
        out_specs=pl.BlockSpec((block_rows, h_out), lambda i: (i, 0)),
        out_shape=jax.ShapeDtypeStruct((n, h_out), jnp.float32),
    )(x, w1, b1.reshape(1, -1), a.reshape(1, 1), w2, b2.reshape(1, -1))


def _update_body(cx_ref, p0_ref, p1_ref, w1_ref, b1_ref, a_ref, w2_ref, b2_ref,
                 eps_ref, lpw_ref, lpb_ref, z_ref, p_ref):
    aggr = p0_ref[...] + p1_ref[...]
    h0 = (1.0 + eps_ref[0, 0]) * cx_ref[...] + aggr
    h = jnp.dot(h0, w1_ref[...], preferred_element_type=jnp.float32)
    h = _prelu(h + b1_ref[...], a_ref[0, 0])
    z = jnp.dot(h, w2_ref[...], preferred_element_type=jnp.float32) + b2_ref[...]
    z_ref[...] = z
    logits = (
        jnp.dot(z, lpw_ref[...], preferred_element_type=jnp.float32) + lpb_ref[...]
    )
    m = jnp.max(logits, axis=1, keepdims=True)
    e = jnp.exp(logits - m)
    p_ref[...] = e / jnp.sum(e, axis=1, keepdims=True)


def _update_pallas(cx, part0, part1, w1, b1, a, w2, b2, eps, lpw, lpb, block_rows):
    n, d = cx.shape
    c = lpw.shape[1]
    grid = n // block_rows
    full = lambda shape: pl.BlockSpec(shape, lambda i: (0, 0))
    row_blk = pl.BlockSpec((block_rows, d), lambda i: (i, 0))
    return pl.pallas_call(
        _update_body,
        grid=(grid,),
        in_specs=[
            row_blk, row_blk, row_blk,
            full(w1.shape), full((1, w1.shape[1])), full((1, 1)),
            full(w2.shape), full((1, w2.shape[1])),
            full((1, 1)),
            full(lpw.shape), full((1, c)),
        ],
        out_specs=[
            pl.BlockSpec((block_rows, d), lambda i: (i, 0)),
            pl.BlockSpec((block_rows, c), lambda i: (i, 0)),
        ],
        out_shape=[
            jax.ShapeDtypeStruct((n, d), jnp.float32),
            jax.ShapeDtypeStruct((n, c), jnp.float32),
        ],
    )(cx, part0, part1, w1, b1.reshape(1, -1), a.reshape(1, 1), w2,
      b2.reshape(1, -1), eps.reshape(1, 1), lpw, lpb.reshape(1, -1))


def _gram_body(zi_ref, zj_ref, o_ref):
    o_ref[...] = lax.dot_general(
        zi_ref[...], zj_ref[...], (((1,), (1,)), ((), ())),
        preferred_element_type=jnp.float32,
    )


def _gram_pallas(z, block_rows):
    n, d = z.shape
    grid = pl.cdiv(n, block_rows)
    return pl.pallas_call(
        _gram_body,
        grid=(grid, grid),
        in_specs=[
            pl.BlockSpec((block_rows, d), lambda i, j: (i, 0)),
            pl.BlockSpec((block_rows, d), lambda i, j: (j, 0)),
        ],
        out_specs=pl.BlockSpec((block_rows, block_rows), lambda i, j: (i, j)),
        out_shape=jax.ShapeDtypeStruct((n, n), jnp.float32),
    )(z, z)


def _make_sc_aggregate(n, e, d, chunk):
    info = plsc.get_sparse_core_info()
    nc, ns = info.num_cores, info.num_subcores
    nw = nc * ns
    assert e % nw == 0
    epw = e // nw
    assert epw % chunk == 0 and chunk % 8 == 0 and chunk <= 128
    nchunk = epw // chunk
    rpt = (n // ns) // 8 * 8
    rlast = n - rpt * (ns - 1)
    lanes = 16
    assert d % lanes == 0

    mesh = plsc.VectorSubcoreMesh(core_axis_name="c", subcore_axis_name="s")

    @functools.partial(
        pl.kernel,
        out_type=jax.ShapeDtypeStruct((nc, n, d), jnp.float32),
        mesh=mesh,
        scratch_types=[
            pltpu.VMEM((chunk,), jnp.int32),
            pltpu.VMEM((chunk,), jnp.int32),
            pltpu.VMEM((chunk, d), jnp.float32),
            pltpu.VMEM((chunk, d), jnp.float32),
            pltpu.VMEM_SHARED((n, d), jnp.float32),
            pltpu.SemaphoreType.DMA,
        ],
    )
    def sc_aggregate(cx_hbm, ce_hbm, src_hbm, dst_hbm, zeros_hbm, out_hbm,
                     sidx, didx, xrows, erows, acc, sem):
        c = lax.axis_index("c")
        s = lax.axis_index("s")
        wid = c * ns + s

        @pl.when(s == 0)
        def _():
            pltpu.sync_copy(zeros_hbm, acc)

        plsc.subcore_barrier()

        base = wid * epw

        def chunk_body(i, carry):
            off = base + i * chunk
            pltpu.sync_copy(src_hbm.at[pl.ds(off, chunk)], sidx)
            pltpu.sync_copy(dst_hbm.at[pl.ds(off, chunk)], didx)
            pltpu.async_copy(cx_hbm.at[sidx], xrows, sem).wait()
            pltpu.sync_copy(ce_hbm.at[pl.ds(off, chunk)], erows)

            def row_body(r, carry2):
                for cc in range(d // lanes):
                    sl = pl.ds(cc * lanes, lanes)
                    xrows[r, sl] = jnp.maximum(xrows[r, sl] + erows[r, sl], 0.0)
                return carry2

            lax.fori_loop(0, chunk, row_body, 0)
            pltpu.sync_copy(xrows, acc.at[didx], add=True)
            return carry

        lax.fori_loop(0, nchunk, chunk_body, 0)
        plsc.subcore_barrier()

        @pl.when(s < ns - 1)
        def _():
            pltpu.sync_copy(acc.at[pl.ds(s * rpt, rpt)],
                            out_hbm.at[c, pl.ds(s * rpt, rpt)])

        @pl.when(s == ns - 1)
        def _():
            pltpu.sync_copy(acc.at[pl.ds(s * rpt, rlast)],
                            out_hbm.at[c, pl.ds(s * rpt, rlast)])

    return sc_aggregate


def kernel(x, edge_attr, ne_W1, ne_b1, ne_a, ne_W2, ne_b2, ee_W1, ee_b1, ee_a,
           ee_W2, ee_b2, pr_W1, pr_b1, pr_a, pr_W2, pr_b2, eps, lp_W, lp_b,
           edge_index):
    n, d = x.shape
    e = edge_attr.shape[0]

    coded_x = _mlp_pallas(x, ne_W1, ne_b1, ne_a, ne_W2, ne_b2, block_rows=1000)
    coded_e = _mlp_pallas(edge_attr, ee_W1, ee_b1, ee_a, ee_W2, ee_b2,
                          block_rows=4000)

    src = edge_index[0]
    dst = edge_index[1]
    zeros = jnp.zeros((n, d), jnp.float32)
    parts = _make_sc_aggregate(n, e, d, chunk=80)(coded_x, coded_e, src, dst,
                                                  zeros)

    z, p = _update_pallas(coded_x, parts[0], parts[1], pr_W1, pr_b1, pr_a,
                          pr_W2, pr_b2, eps, lp_W, lp_b, block_rows=1000)
    a_hat = _gram_pallas(z, block_rows=1024)
    return (a_hat, p, z)


# speedup vs baseline: 3.1847x; 1.5287x over previous
import functools

import jax
import jax.numpy as jnp
from jax import lax
from jax.experimental import pallas as pl
from jax.experimental.pallas import tpu as pltpu
from jax.experimental.pallas import tpu_sc as plsc


def _prelu(h, a):
    return jnp.where(h >= 0, h, a * h)


def _mlp_body(x_ref, w1_ref, b1_ref, a_ref, w2_ref, b2_ref, o_ref):
    h = jnp.dot(x_ref[...], w1_ref[...], preferred_element_type=jnp.float32)
    h = _prelu(h + b1_ref[...], a_ref[0, 0])
    o_ref[...] = (
        jnp.dot(h, w2_ref[...], preferred_element_type=jnp.float32) + b2_ref[...]
    )


def _mlp_pallas(x, w1, b1, a, w2, b2, block_rows):
    n, _ = x.shape
    h_out = w2.shape[1]
    grid = n // block_rows
    full = lambda shape: pl.BlockSpec(shape, lambda i: (0, 0))
    return pl.pallas_call(
        _mlp_body,
        grid=(grid,),
        in_specs=[
            pl.BlockSpec((block_rows, x.shape[1]), lambda i: (i, 0)),
            full(w1.shape),
            full((1, w1.shape[1])),
            full((1, 1)),
            full(w2.shape),
            full((1, h_out)),
        ],
        out_specs=pl.BlockSpec((block_rows, h_out), lambda i: (i, 0)),
        out_shape=jax.ShapeDtypeStruct((n, h_out), jnp.float32),
    )(x, w1, b1.reshape(1, -1), a.reshape(1, 1), w2, b2.reshape(1, -1))


def _update_body(cx_ref, p0_ref, p1_ref, w1_ref, b1_ref, a_ref, w2_ref, b2_ref,
                 eps_ref, lpw_ref, lpb_ref, z_ref, p_ref):
    aggr = p0_ref[...] + p1_ref[...]
    h0 = (1.0 + eps_ref[0, 0]) * cx_ref[...] + aggr
    h = jnp.dot(h0, w1_ref[...], preferred_element_type=jnp.float32)
    h = _prelu(h + b1_ref[...], a_ref[0, 0])
    z = jnp.dot(h, w2_ref[...], preferred_element_type=jnp.float32) + b2_ref[...]
    z_ref[...] = z
    logits = (
        jnp.dot(z, lpw_ref[...], preferred_element_type=jnp.float32) + lpb_ref[...]
    )
    m = jnp.max(logits, axis=1, keepdims=True)
    e = jnp.exp(logits - m)
    p_ref[...] = e / jnp.sum(e, axis=1, keepdims=True)


def _update_pallas(cx, part0, part1, w1, b1, a, w2, b2, eps, lpw, lpb, block_rows):
    n, d = cx.shape
    c = lpw.shape[1]
    grid = n // block_rows
    full = lambda shape: pl.BlockSpec(shape, lambda i: (0, 0))
    row_blk = pl.BlockSpec((block_rows, d), lambda i: (i, 0))
    return pl.pallas_call(
        _update_body,
        grid=(grid,),
        in_specs=[
            row_blk, row_blk, row_blk,
            full(w1.shape), full((1, w1.shape[1])), full((1, 1)),
            full(w2.shape), full((1, w2.shape[1])),
            full((1, 1)),
            full(lpw.shape), full((1, c)),
        ],
        out_specs=[
            pl.BlockSpec((block_rows, d), lambda i: (i, 0)),
            pl.BlockSpec((block_rows, c), lambda i: (i, 0)),
        ],
        out_shape=[
            jax.ShapeDtypeStruct((n, d), jnp.float32),
            jax.ShapeDtypeStruct((n, c), jnp.float32),
        ],
    )(cx, part0, part1, w1, b1.reshape(1, -1), a.reshape(1, 1), w2,
      b2.reshape(1, -1), eps.reshape(1, 1), lpw, lpb.reshape(1, -1))


def _gram_body(zi_ref, zj_ref, o_ref):
    o_ref[...] = lax.dot_general(
        zi_ref[...], zj_ref[...], (((1,), (1,)), ((), ())),
        preferred_element_type=jnp.float32,
    )


def _gram_pallas(z, block_rows):
    n, d = z.shape
    grid = pl.cdiv(n, block_rows)
    return pl.pallas_call(
        _gram_body,
        grid=(grid, grid),
        in_specs=[
            pl.BlockSpec((block_rows, d), lambda i, j: (i, 0)),
            pl.BlockSpec((block_rows, d), lambda i, j: (j, 0)),
        ],
        out_specs=pl.BlockSpec((block_rows, block_rows), lambda i, j: (i, j)),
        out_shape=jax.ShapeDtypeStruct((n, n), jnp.float32),
    )(z, z)


def _make_sc_aggregate(n, e, d, chunk):
    info = plsc.get_sparse_core_info()
    nc, ns = info.num_cores, info.num_subcores
    nw = nc * ns
    assert e % nw == 0
    epw = e // nw
    assert epw % chunk == 0 and chunk % 8 == 0 and chunk <= 128
    nchunk = epw // chunk
    rpt = (n // ns) // 8 * 8
    rlast = n - rpt * (ns - 1)
    lanes = 16
    assert d % lanes == 0

    mesh = plsc.VectorSubcoreMesh(core_axis_name="c", subcore_axis_name="s")

    @functools.partial(
        pl.kernel,
        out_type=jax.ShapeDtypeStruct((nc, n, d), jnp.float32),
        mesh=mesh,
        scratch_types=[
            pltpu.VMEM((chunk,), jnp.int32),
            pltpu.VMEM((chunk,), jnp.int32),
            pltpu.VMEM((chunk,), jnp.int32),
            pltpu.VMEM((chunk,), jnp.int32),
            pltpu.VMEM((chunk, d), jnp.float32),
            pltpu.VMEM((chunk, d), jnp.float32),
            pltpu.VMEM((chunk, d), jnp.float32),
            pltpu.VMEM((chunk, d), jnp.float32),
            pltpu.VMEM_SHARED((n, d), jnp.float32),
            pltpu.SemaphoreType.DMA,
            pltpu.SemaphoreType.DMA,
            pltpu.SemaphoreType.DMA,
            pltpu.SemaphoreType.DMA,
        ],
    )
    def sc_aggregate(cx_hbm, ce_hbm, src_hbm, dst_hbm, zeros_hbm, out_hbm,
                     sidx0, sidx1, didx0, didx1, xr0, xr1, er0, er1, acc,
                     isem0, isem1, dsem0, dsem1):
        c = lax.axis_index("c")
        s = lax.axis_index("s")
        wid = c * ns + s

        sidx = (sidx0, sidx1)
        didx = (didx0, didx1)
        xr = (xr0, xr1)
        er = (er0, er1)
        isem = (isem0, isem1)
        dsem = (dsem0, dsem1)

        @pl.when(s == 0)
        def _():
            pltpu.sync_copy(zeros_hbm, acc)

        plsc.subcore_barrier()

        base = wid * epw

        def issue_idx(j, b):
            off = base + j * chunk
            pltpu.async_copy(src_hbm.at[pl.ds(off, chunk)], sidx[b], isem[b])

        def drain_idx(b):
            pltpu.make_async_copy(src_hbm.at[pl.ds(0, chunk)], sidx[b],
                                  isem[b]).wait()

        def issue_data(j, b):
            off = base + j * chunk
            pltpu.async_copy(cx_hbm.at[sidx[b]], xr[b], dsem[b])
            pltpu.async_copy(ce_hbm.at[pl.ds(off, chunk)], er[b], dsem[b])
            pltpu.async_copy(dst_hbm.at[pl.ds(off, chunk)], didx[b], dsem[b])

        def drain_data(b):
            pltpu.make_async_copy(cx_hbm.at[sidx[b]], xr[b], dsem[b]).wait()
            pltpu.make_async_copy(ce_hbm.at[pl.ds(0, chunk)], er[b],
                                  dsem[b]).wait()
            pltpu.make_async_copy(dst_hbm.at[pl.ds(0, chunk)], didx[b],
                                  dsem[b]).wait()

        def compute(b):
            xb, eb = xr[b], er[b]

            def row_body(r, carry2):
                for cc in range(d // lanes):
                    sl = pl.ds(cc * lanes, lanes)
                    xb[r, sl] = jnp.maximum(xb[r, sl] + eb[r, sl], 0.0)
                return carry2

            lax.fori_loop(0, chunk, row_body, 0)

        issue_idx(0, 0)
        drain_idx(0)
        issue_data(0, 0)
        if nchunk > 1:
            issue_idx(1, 1)

        def step(j, b, b2):
            @pl.when(j + 1 < nchunk)
            def _():
                drain_idx(b2)
                issue_data(j + 1, b2)

            drain_data(b)

            @pl.when(j + 2 < nchunk)
            def _():
                issue_idx(j + 2, b)

            compute(b)
            pltpu.sync_copy(xr[b], acc.at[didx[b]], add=True)

        def pair_body(p, carry):
            step(2 * p, 0, 1)
            step(2 * p + 1, 1, 0)
            return carry

        lax.fori_loop(0, nchunk // 2, pair_body, 0)
        if nchunk % 2 == 1:
            step(nchunk - 1, 0, 1)
        plsc.subcore_barrier()

        @pl.when(s < ns - 1)
        def _():
            pltpu.sync_copy(acc.at[pl.ds(s * rpt, rpt)],
                            out_hbm.at[c, pl.ds(s * rpt, rpt)])

        @pl.when(s == ns - 1)
        def _():
            pltpu.sync_copy(acc.at[pl.ds(s * rpt, rlast)],
                            out_hbm.at[c, pl.ds(s * rpt, rlast)])

    return sc_aggregate


def kernel(x, edge_attr, ne_W1, ne_b1, ne_a, ne_W2, ne_b2, ee_W1, ee_b1, ee_a,
           ee_W2, ee_b2, pr_W1, pr_b1, pr_a, pr_W2, pr_b2, eps, lp_W, lp_b,
           edge_index):
    n, d = x.shape
    e = edge_attr.shape[0]

    coded_x = _mlp_pallas(x, ne_W1, ne_b1, ne_a, ne_W2, ne_b2, block_rows=1000)
    coded_e = _mlp_pallas(edge_attr, ee_W1, ee_b1, ee_a, ee_W2, ee_b2,
                          block_rows=4000)

    src = edge_index[0]
    dst = edge_index[1]
    zeros = jnp.zeros((n, d), jnp.float32)
    parts = _make_sc_aggregate(n, e, d, chunk=80)(coded_x, coded_e, src, dst,
                                                  zeros)

    z, p = _update_pallas(coded_x, parts[0], parts[1], pr_W1, pr_b1, pr_a,
                          pr_W2, pr_b2, eps, lp_W, lp_b, block_rows=1000)
    a_hat = _gram_pallas(z, block_rows=1024)
    return (a_hat, p, z)


# speedup vs baseline: 3.4063x; 1.0696x over previous
import functools

import jax
import jax.numpy as jnp
from jax import lax
from jax.experimental import pallas as pl
from jax.experimental.pallas import tpu as pltpu
from jax.experimental.pallas import tpu_sc as plsc


def _prelu(h, a):
    return jnp.where(h >= 0, h, a * h)


def _mlp_body(x_ref, w1_ref, b1_ref, a_ref, w2_ref, b2_ref, o_ref):
    h = jnp.dot(x_ref[...], w1_ref[...], preferred_element_type=jnp.float32)
    h = _prelu(h + b1_ref[...], a_ref[0, 0])
    o_ref[...] = (
        jnp.dot(h, w2_ref[...], preferred_element_type=jnp.float32) + b2_ref[...]
    )


def _mlp_pallas(x, w1, b1, a, w2, b2, block_rows):
    n, _ = x.shape
    h_out = w2.shape[1]
    grid = n // block_rows
    full = lambda shape: pl.BlockSpec(shape, lambda i: (0, 0))
    return pl.pallas_call(
        _mlp_body,
        grid=(grid,),
        in_specs=[
            pl.BlockSpec((block_rows, x.shape[1]), lambda i: (i, 0)),
            full(w1.shape),
            full((1, w1.shape[1])),
            full((1, 1)),
            full(w2.shape),
            full((1, h_out)),
        ],
        out_specs=pl.BlockSpec((block_rows, h_out), lambda i: (i, 0)),
        out_shape=jax.ShapeDtypeStruct((n, h_out), jnp.float32),
    )(x, w1, b1.reshape(1, -1), a.reshape(1, 1), w2, b2.reshape(1, -1))


def _update_body(cx_ref, p0_ref, p1_ref, w1_ref, b1_ref, a_ref, w2_ref, b2_ref,
                 eps_ref, lpw_ref, lpb_ref, z_ref, p_ref):
    aggr = p0_ref[...] + p1_ref[...]
    h0 = (1.0 + eps_ref[0, 0]) * cx_ref[...] + aggr
    h = jnp.dot(h0, w1_ref[...], preferred_element_type=jnp.float32)
    h = _prelu(h + b1_ref[...], a_ref[0, 0])
    z = jnp.dot(h, w2_ref[...], preferred_element_type=jnp.float32) + b2_ref[...]
    z_ref[...] = z
    logits = (
        jnp.dot(z, lpw_ref[...], preferred_element_type=jnp.float32) + lpb_ref[...]
    )
    m = jnp.max(logits, axis=1, keepdims=True)
    e = jnp.exp(logits - m)
    p_ref[...] = e / jnp.sum(e, axis=1, keepdims=True)


def _update_pallas(cx, part0, part1, w1, b1, a, w2, b2, eps, lpw, lpb, block_rows):
    n, d = cx.shape
    c = lpw.shape[1]
    grid = n // block_rows
    full = lambda shape: pl.BlockSpec(shape, lambda i: (0, 0))
    row_blk = pl.BlockSpec((block_rows, d), lambda i: (i, 0))
    return pl.pallas_call(
        _update_body,
        grid=(grid,),
        in_specs=[
            row_blk, row_blk, row_blk,
            full(w1.shape), full((1, w1.shape[1])), full((1, 1)),
            full(w2.shape), full((1, w2.shape[1])),
            full((1, 1)),
            full(lpw.shape), full((1, c)),
        ],
        out_specs=[
            pl.BlockSpec((block_rows, d), lambda i: (i, 0)),
            pl.BlockSpec((block_rows, c), lambda i: (i, 0)),
        ],
        out_shape=[
            jax.ShapeDtypeStruct((n, d), jnp.float32),
            jax.ShapeDtypeStruct((n, c), jnp.float32),
        ],
    )(cx, part0, part1, w1, b1.reshape(1, -1), a.reshape(1, 1), w2,
      b2.reshape(1, -1), eps.reshape(1, 1), lpw, lpb.reshape(1, -1))


def _gram_body(zi_ref, zall_ref, o_ref):
    o_ref[...] = lax.dot_general(
        zi_ref[...], zall_ref[...], (((1,), (1,)), ((), ())),
        preferred_element_type=jnp.float32,
    )


def _gram_pallas(z, block_rows):
    n, d = z.shape
    grid = n // block_rows
    return pl.pallas_call(
        _gram_body,
        grid=(grid,),
        in_specs=[
            pl.BlockSpec((block_rows, d), lambda i: (i, 0)),
            pl.BlockSpec((n, d), lambda i: (0, 0)),
        ],
        out_specs=pl.BlockSpec((block_rows, n), lambda i: (i, 0)),
        out_shape=jax.ShapeDtypeStruct((n, n), jnp.float32),
    )(z, z)


def _make_sc_aggregate(n, e, d, chunk):
    info = plsc.get_sparse_core_info()
    nc, ns = info.num_cores, info.num_subcores
    nw = nc * ns
    assert e % nw == 0
    epw = e // nw
    assert epw % chunk == 0 and chunk % 8 == 0 and chunk <= 128
    nchunk = epw // chunk
    rpt = (n // ns) // 8 * 8
    rlast = n - rpt * (ns - 1)
    lanes = 16
    assert d % lanes == 0

    mesh = plsc.VectorSubcoreMesh(core_axis_name="c", subcore_axis_name="s")

    @functools.partial(
        pl.kernel,
        out_type=jax.ShapeDtypeStruct((nc, n, d), jnp.float32),
        mesh=mesh,
        scratch_types=[
            pltpu.VMEM((chunk,), jnp.int32),
            pltpu.VMEM((chunk,), jnp.int32),
            pltpu.VMEM((chunk,), jnp.int32),
            pltpu.VMEM((chunk,), jnp.int32),
            pltpu.VMEM((chunk, d), jnp.float32),
            pltpu.VMEM((chunk, d), jnp.float32),
            pltpu.VMEM((chunk, d), jnp.float32),
            pltpu.VMEM((chunk, d), jnp.float32),
            pltpu.VMEM_SHARED((n, d), jnp.float32),
            pltpu.SemaphoreType.DMA,
            pltpu.SemaphoreType.DMA,
            pltpu.SemaphoreType.DMA,
            pltpu.SemaphoreType.DMA,
        ],
    )
    def sc_aggregate(cx_hbm, ce_hbm, src_hbm, dst_hbm, zeros_hbm, out_hbm,
                     sidx0, sidx1, didx0, didx1, xr0, xr1, er0, er1, acc,
                     isem0, isem1, dsem0, dsem1):
        c = lax.axis_index("c")
        s = lax.axis_index("s")
        wid = c * ns + s

        sidx = (sidx0, sidx1)
        didx = (didx0, didx1)
        xr = (xr0, xr1)
        er = (er0, er1)
        isem = (isem0, isem1)
        dsem = (dsem0, dsem1)

        @pl.when(s == 0)
        def _():
            pltpu.sync_copy(zeros_hbm, acc)

        plsc.subcore_barrier()

        base = wid * epw

        def issue_idx(j, b):
            off = base + j * chunk
            pltpu.async_copy(src_hbm.at[pl.ds(off, chunk)], sidx[b], isem[b])

        def drain_idx(b):
            pltpu.make_async_copy(src_hbm.at[pl.ds(0, chunk)], sidx[b],
                                  isem[b]).wait()

        def issue_data(j, b):
            off = base + j * chunk
            pltpu.async_copy(cx_hbm.at[sidx[b]], xr[b], dsem[b])
            pltpu.async_copy(ce_hbm.at[pl.ds(off, chunk)], er[b], dsem[b])
            pltpu.async_copy(dst_hbm.at[pl.ds(off, chunk)], didx[b], dsem[b])

        def drain_data(b):
            pltpu.make_async_copy(cx_hbm.at[sidx[b]], xr[b], dsem[b]).wait()
            pltpu.make_async_copy(ce_hbm.at[pl.ds(0, chunk)], er[b],
                                  dsem[b]).wait()
            pltpu.make_async_copy(dst_hbm.at[pl.ds(0, chunk)], didx[b],
                                  dsem[b]).wait()

        def compute(b):
            xb, eb = xr[b], er[b]

            def row_body(r, carry2):
                for cc in range(d // lanes):
                    sl = pl.ds(cc * lanes, lanes)
                    xb[r, sl] = jnp.maximum(xb[r, sl] + eb[r, sl], 0.0)
                return carry2

            lax.fori_loop(0, chunk, row_body, 0)

        issue_idx(0, 0)
        drain_idx(0)
        issue_data(0, 0)
        if nchunk > 1:
            issue_idx(1, 1)

        def step(j, b, b2):
            @pl.when(j + 1 < nchunk)
            def _():
                drain_idx(b2)
                issue_data(j + 1, b2)

            drain_data(b)

            @pl.when(j + 2 < nchunk)
            def _():
                issue_idx(j + 2, b)

            compute(b)
            pltpu.sync_copy(xr[b], acc.at[didx[b]], add=True)

        def pair_body(p, carry):
            step(2 * p, 0, 1)
            step(2 * p + 1, 1, 0)
            return carry

        lax.fori_loop(0, nchunk // 2, pair_body, 0)
        if nchunk % 2 == 1:
            step(nchunk - 1, 0, 1)
        plsc.subcore_barrier()

        @pl.when(s < ns - 1)
        def _():
            pltpu.sync_copy(acc.at[pl.ds(s * rpt, rpt)],
                            out_hbm.at[c, pl.ds(s * rpt, rpt)])

        @pl.when(s == ns - 1)
        def _():
            pltpu.sync_copy(acc.at[pl.ds(s * rpt, rlast)],
                            out_hbm.at[c, pl.ds(s * rpt, rlast)])

    return sc_aggregate


def kernel(x, edge_attr, ne_W1, ne_b1, ne_a, ne_W2, ne_b2, ee_W1, ee_b1, ee_a,
           ee_W2, ee_b2, pr_W1, pr_b1, pr_a, pr_W2, pr_b2, eps, lp_W, lp_b,
           edge_index):
    n, d = x.shape
    e = edge_attr.shape[0]

    coded_x = _mlp_pallas(x, ne_W1, ne_b1, ne_a, ne_W2, ne_b2, block_rows=1000)
    coded_e = _mlp_pallas(edge_attr, ee_W1, ee_b1, ee_a, ee_W2, ee_b2,
                          block_rows=4000)

    src = edge_index[0]
    dst = edge_index[1]
    zeros = jnp.zeros((n, d), jnp.float32)
    parts = _make_sc_aggregate(n, e, d, chunk=80)(coded_x, coded_e, src, dst,
                                                  zeros)

    z, p = _update_pallas(coded_x, parts[0], parts[1], pr_W1, pr_b1, pr_a,
                          pr_W2, pr_b2, eps, lp_W, lp_b, block_rows=1000)
    a_hat = _gram_pallas(z, block_rows=400)
    return (a_hat, p, z)


# speedup vs baseline: 3.7423x; 1.0986x over previous
import functools

import jax
import jax.numpy as jnp
from jax import lax
from jax.experimental import pallas as pl
from jax.experimental.pallas import tpu as pltpu
from jax.experimental.pallas import tpu_sc as plsc


def _prelu(h, a):
    return jnp.where(h >= 0, h, a * h)


def _mlp_body(x_ref, w1_ref, b1_ref, a_ref, w2_ref, b2_ref, o_ref):
    h = jnp.dot(x_ref[...], w1_ref[...], preferred_element_type=jnp.float32)
    h = _prelu(h + b1_ref[...], a_ref[0, 0])
    out = jnp.dot(h, w2_ref[...], preferred_element_type=jnp.float32) + b2_ref[...]
    o_ref[...] = out.astype(o_ref.dtype)


def _mlp_pallas(x, w1, b1, a, w2, b2, block_rows, out_dtype=jnp.float32):
    n, _ = x.shape
    h_out = w2.shape[1]
    grid = n // block_rows
    full = lambda shape: pl.BlockSpec(shape, lambda i: (0, 0))
    return pl.pallas_call(
        _mlp_body,
        grid=(grid,),
        in_specs=[
            pl.BlockSpec((block_rows, x.shape[1]), lambda i: (i, 0)),
            full(w1.shape),
            full((1, w1.shape[1])),
            full((1, 1)),
            full(w2.shape),
            full((1, h_out)),
        ],
        out_specs=pl.BlockSpec((block_rows, h_out), lambda i: (i, 0)),
        out_shape=jax.ShapeDtypeStruct((n, h_out), out_dtype),
    )(x, w1, b1.reshape(1, -1), a.reshape(1, 1), w2, b2.reshape(1, -1))


def _update_body(cx_ref, pa_ref, pb_ref, w1_ref, b1_ref, a_ref, w2_ref, b2_ref,
                 eps_ref, lpw_ref, lpb_ref, z_ref, p_ref):
    aggr = (pa_ref[0] + pa_ref[1]) + (pb_ref[0] + pb_ref[1])
    h0 = (1.0 + eps_ref[0, 0]) * cx_ref[...] + aggr
    h = jnp.dot(h0, w1_ref[...], preferred_element_type=jnp.float32)
    h = _prelu(h + b1_ref[...], a_ref[0, 0])
    z = jnp.dot(h, w2_ref[...], preferred_element_type=jnp.float32) + b2_ref[...]
    z_ref[...] = z
    logits = (
        jnp.dot(z, lpw_ref[...], preferred_element_type=jnp.float32) + lpb_ref[...]
    )
    m = jnp.max(logits, axis=1, keepdims=True)
    e = jnp.exp(logits - m)
    p_ref[...] = e / jnp.sum(e, axis=1, keepdims=True)


def _update_pallas(cx, parts_a, parts_b, w1, b1, a, w2, b2, eps, lpw, lpb,
                   block_rows):
    n, d = cx.shape
    c = lpw.shape[1]
    grid = n // block_rows
    full = lambda shape: pl.BlockSpec(shape, lambda i: (0, 0))
    row_blk = pl.BlockSpec((block_rows, d), lambda i: (i, 0))
    part_blk = pl.BlockSpec((2, block_rows, d), lambda i: (0, i, 0))
    return pl.pallas_call(
        _update_body,
        grid=(grid,),
        in_specs=[
            row_blk, part_blk, part_blk,
            full(w1.shape), full((1, w1.shape[1])), full((1, 1)),
            full(w2.shape), full((1, w2.shape[1])),
            full((1, 1)),
            full(lpw.shape), full((1, c)),
        ],
        out_specs=[
            pl.BlockSpec((block_rows, d), lambda i: (i, 0)),
            pl.BlockSpec((block_rows, c), lambda i: (i, 0)),
        ],
        out_shape=[
            jax.ShapeDtypeStruct((n, d), jnp.float32),
            jax.ShapeDtypeStruct((n, c), jnp.float32),
        ],
    )(cx, parts_a, parts_b, w1, b1.reshape(1, -1), a.reshape(1, 1), w2,
      b2.reshape(1, -1), eps.reshape(1, 1), lpw, lpb.reshape(1, -1))


def _gram_body(zi_ref, zall_ref, o_ref):
    o_ref[...] = lax.dot_general(
        zi_ref[...], zall_ref[...], (((1,), (1,)), ((), ())),
        preferred_element_type=jnp.float32,
    )


def _gram_pallas(z, block_rows):
    n, d = z.shape
    grid = n // block_rows
    return pl.pallas_call(
        _gram_body,
        grid=(grid,),
        in_specs=[
            pl.BlockSpec((block_rows, d), lambda i: (i, 0)),
            pl.BlockSpec((n, d), lambda i: (0, 0)),
        ],
        out_specs=pl.BlockSpec((block_rows, n), lambda i: (i, 0)),
        out_shape=jax.ShapeDtypeStruct((n, n), jnp.float32),
    )(z, z)


def _make_sc_aggregate(n, e, d, chunk, idx_offset=0):
    info = plsc.get_sparse_core_info()
    nc, ns = info.num_cores, info.num_subcores
    nw = nc * ns
    assert e % nw == 0 and idx_offset % 8 == 0
    epw = e // nw
    assert epw % chunk == 0 and chunk % 8 == 0 and chunk <= 128
    nchunk = epw // chunk
    rpt = (n // ns) // 8 * 8
    rlast = n - rpt * (ns - 1)
    lanes = 16
    assert d % lanes == 0

    mesh = plsc.VectorSubcoreMesh(core_axis_name="c", subcore_axis_name="s")

    @functools.partial(
        pl.kernel,
        out_type=jax.ShapeDtypeStruct((nc, n, d), jnp.float32),
        mesh=mesh,
        scratch_types=[
            pltpu.VMEM((chunk,), jnp.int32),
            pltpu.VMEM((chunk,), jnp.int32),
            pltpu.VMEM((chunk,), jnp.int32),
            pltpu.VMEM((chunk,), jnp.int32),
            pltpu.VMEM((chunk, d), jnp.float32),
            pltpu.VMEM((chunk, d), jnp.float32),
            pltpu.VMEM((chunk, d), jnp.float32),
            pltpu.VMEM((chunk, d), jnp.float32),
            pltpu.VMEM_SHARED((n, d), jnp.float32),
            pltpu.SemaphoreType.DMA,
            pltpu.SemaphoreType.DMA,
            pltpu.SemaphoreType.DMA,
            pltpu.SemaphoreType.DMA,
        ],
    )
    def sc_aggregate(cx_hbm, ce_hbm, src_hbm, dst_hbm, zeros_hbm, out_hbm,
                     sidx0, sidx1, didx0, didx1, xr0, xr1, er0, er1, acc,
                     isem0, isem1, dsem0, dsem1):
        c = lax.axis_index("c")
        s = lax.axis_index("s")
        wid = c * ns + s

        sidx = (sidx0, sidx1)
        didx = (didx0, didx1)
        xr = (xr0, xr1)
        er = (er0, er1)
        isem = (isem0, isem1)
        dsem = (dsem0, dsem1)

        @pl.when(s == 0)
        def _():
            pltpu.sync_copy(zeros_hbm, acc)

        plsc.subcore_barrier()

        base = idx_offset + wid * epw
        base_ce = wid * epw

        def issue_idx(j, b):
            off = base + j * chunk
            pltpu.async_copy(src_hbm.at[pl.ds(off, chunk)], sidx[b], isem[b])

        def drain_idx(b):
            pltpu.make_async_copy(src_hbm.at[pl.ds(0, chunk)], sidx[b],
                                  isem[b]).wait()

        def issue_data(j, b):
            off = base + j * chunk
            off_ce = base_ce + j * chunk
            pltpu.async_copy(cx_hbm.at[sidx[b]], xr[b], dsem[b])
            pltpu.async_copy(ce_hbm.at[pl.ds(off_ce, chunk)], er[b], dsem[b])
            pltpu.async_copy(dst_hbm.at[pl.ds(off, chunk)], didx[b], dsem[b])

        def drain_data(b):
            pltpu.make_async_copy(cx_hbm.at[sidx[b]], xr[b], dsem[b]).wait()
            pltpu.make_async_copy(ce_hbm.at[pl.ds(0, chunk)], er[b],
                                  dsem[b]).wait()
            pltpu.make_async_copy(dst_hbm.at[pl.ds(0, chunk)], didx[b],
                                  dsem[b]).wait()

        def compute(b):
            xb, eb = xr[b], er[b]

            def row_body(r, carry2):
                for cc in range(d // lanes):
                    sl = pl.ds(cc * lanes, lanes)
                    xb[r, sl] = jnp.maximum(xb[r, sl] + eb[r, sl], 0.0)
                return carry2

            lax.fori_loop(0, chunk, row_body, 0)

        issue_idx(0, 0)
        drain_idx(0)
        issue_data(0, 0)
        if nchunk > 1:
            issue_idx(1, 1)

        def step(j, b, b2):
            @pl.when(j + 1 < nchunk)
            def _():
                drain_idx(b2)
                issue_data(j + 1, b2)

            drain_data(b)

            @pl.when(j + 2 < nchunk)
            def _():
                issue_idx(j + 2, b)

            compute(b)
            pltpu.sync_copy(xr[b], acc.at[didx[b]], add=True)

        def pair_body(p, carry):
            step(2 * p, 0, 1)
            step(2 * p + 1, 1, 0)
            return carry

        lax.fori_loop(0, nchunk // 2, pair_body, 0)
        if nchunk % 2 == 1:
            step(nchunk - 1, 0, 1)
        plsc.subcore_barrier()

        @pl.when(s < ns - 1)
        def _():
            pltpu.sync_copy(acc.at[pl.ds(s * rpt, rpt)],
                            out_hbm.at[c, pl.ds(s * rpt, rpt)])

        @pl.when(s == ns - 1)
        def _():
            pltpu.sync_copy(acc.at[pl.ds(s * rpt, rlast)],
                            out_hbm.at[c, pl.ds(s * rpt, rlast)])

    return sc_aggregate


def kernel(x, edge_attr, ne_W1, ne_b1, ne_a, ne_W2, ne_b2, ee_W1, ee_b1, ee_a,
           ee_W2, ee_b2, pr_W1, pr_b1, pr_a, pr_W2, pr_b2, eps, lp_W, lp_b,
           edge_index):
    n, d = x.shape
    e = edge_attr.shape[0]

    coded_x = _mlp_pallas(x, ne_W1, ne_b1, ne_a, ne_W2, ne_b2, block_rows=1000)

    e_a = 163840
    e_b = e - e_a
    coded_e_a = _mlp_pallas(edge_attr[:e_a], ee_W1, ee_b1, ee_a, ee_W2, ee_b2,
                            block_rows=4096)
    coded_e_b = _mlp_pallas(edge_attr[e_a:], ee_W1, ee_b1, ee_a, ee_W2, ee_b2,
                            block_rows=4880)

    src = edge_index[0]
    dst = edge_index[1]
    zeros = jnp.zeros((n, d), jnp.float32)
    parts_a = _make_sc_aggregate(n, e_a, d, chunk=80)(
        coded_x, coded_e_a, src, dst, zeros)
    parts_b = _make_sc_aggregate(n, e_b, d, chunk=80, idx_offset=e_a)(
        coded_x, coded_e_b, src, dst, zeros)

    z, p = _update_pallas(coded_x, parts_a, parts_b, pr_W1, pr_b1, pr_a,
                          pr_W2, pr_b2, eps, lp_W, lp_b, block_rows=1000)
    a_hat = _gram_pallas(z, block_rows=400)
    return (a_hat, p, z)
